# baseline (speedup 1.0000x reference)
import numpy as np
import jax
import jax.numpy as jnp
from jax import lax
from jax.experimental import pallas as pl
from jax.experimental.pallas import tpu as pltpu
from jax.experimental.pallas import tpu_sc as plsc

_R = 5
_RC = 8
_BANDL = 24
_W = 512
_WP = 528
_HP = 528
_PLANE = _HP * _WP
_BSZ = _BANDL * _WP
_OSZ = _RC * _W
_YSC = 256
_TH = 32
_RPW = _YSC // 8


def _make_taps():
    m = 2.0 * np.pi / 10000.0
    half = np.pi / 10000.0
    cos_half = np.cos(half)
    init_angle = 3.1415926536 / 2.0
    taps = []
    for dy in range(-_R, _R + 1):
        for dx in range(-_R, _R + 1):
            if dy * dy + dx * dx > 20:
                continue
            dist = np.sqrt(float(dy * dy + dx * dx))
            theta = np.arctan2(float(dy), float(dx)) - init_angle
            ang = np.mod(theta, m) - half
            c32 = np.float32(cos_half / np.cos(ang))
            d32 = np.float32(dist)
            need_min = dy * dy + dx * dx <= 12
            off = int((_R - dy) * _WP + (_R - dx))
            taps.append((off, d32, c32, need_min))
    return taps


_TAPS = _make_taps()


def _sc_body(imgp, dp, bok_out, dil_out, dbuf, rcpbuf, ib0, ib1, ib2, obuf,
             dilbuf):
    ibufs = (ib0, ib1, ib2)
    wid = lax.axis_index("s") * 2 + lax.axis_index("c")
    b = wid // 8
    y0 = (wid % 8) * _RPW
    lanes = lax.broadcasted_iota(jnp.int32, (16,), 0)

    def chunk(k, _):
        ys = y0 + k * _RC

        pltpu.sync_copy(dp.at[pl.ds(b * _PLANE + ys * _WP, _BSZ)], dbuf)
        for cc in range(3):
            pltpu.sync_copy(
                imgp.at[pl.ds((b * 3 + cc) * _PLANE + ys * _WP, _BSZ)],
                ibufs[cc])

        def pre(i, _):
            off = pl.multiple_of(i * 16, 16)
            v = dbuf[pl.ds(off, 16)]
            r = jnp.abs(v)
            dbuf[pl.ds(off, 16)] = r
            rcpbuf[pl.ds(off, 16)] = 1.0 / (r * r + jnp.float32(0.2))
            return 0

        lax.fori_loop(0, (_RC + 2 * _R) * _WP // 16, pre, 0, unroll=False)

        def out_row(yo, _):
            @plsc.parallel_loop(0, _W // 16, unroll=2)
            def group(g):
                c0 = g * 16
                base = yo * _WP + c0
                vbase = jnp.full((16,), base, jnp.int32) + lanes
                vb = [vbase + r for r in range(8)]
                accw = jnp.zeros((16,), jnp.float32)
                acc = [jnp.zeros((16,), jnp.float32) for _ in range(3)]
                mr = jnp.full((16,), -1.0, jnp.float32)
                for off, d32, c32, need_min in _TAPS:
                    off8 = (off // 8) * 8
                    sl = pl.ds(off8, _BSZ - off8)
                    vidx = vb[off % 8]
                    rs = plsc.load_gather(dbuf.at[sl], [vidx])
                    t = rs * c32 if c32 != np.float32(1.0) else rs
                    t = t + jnp.float32(0.5)
                    t = t - d32
                    ws = jnp.maximum(t, jnp.float32(0.0))
                    if need_min:
                        ws = jnp.minimum(ws, jnp.float32(1.0))
                    w = ws * plsc.load_gather(rcpbuf.at[sl], [vidx])
                    accw = accw + w
                    for cc in range(3):
                        acc[cc] = acc[cc] + w * plsc.load_gather(
                            ibufs[cc].at[sl], [vidx])
                    mr = jnp.maximum(
                        mr, jnp.where(t > jnp.float32(0.0), rs,
                                      jnp.float32(-1.0)))
                rw = 1.0 / accw
                oix = pl.multiple_of(yo * _W + c0, 16)
                for cc in range(3):
                    obuf[pl.ds(cc * _OSZ + oix, 16)] = acc[cc] * rw
                dilbuf[pl.ds(oix, 16)] = (
                    mr.astype(jnp.int32).astype(jnp.float32))

            return 0

        lax.fori_loop(0, _RC, out_row, 0, unroll=False)

        for cc in range(3):
            pltpu.sync_copy(
                obuf.at[pl.ds(cc * _OSZ, _OSZ)],
                bok_out.at[pl.ds((b * 3 + cc) * (_YSC * _W) + ys * _W, _OSZ)])
        pltpu.sync_copy(
            dilbuf, dil_out.at[pl.ds(b * (_YSC * _W) + ys * _W, _OSZ)])
        return 0

    lax.fori_loop(0, _RPW // _RC, chunk, 0, unroll=False)


def _tc_body(imgp_ref, dp_ref, bokeh_ref, dil_ref):
    th = bokeh_ref.shape[2]
    w_out = bokeh_ref.shape[3]
    y0 = (pl.program_id(1) + _YSC // _TH) * th
    band_h = th + 2 * _R
    d_band = dp_ref[0, 0, pl.ds(y0, band_h), :]
    r = jnp.abs(d_band)
    rcp = 1.0 / (r * r + jnp.float32(0.2))
    fdi = d_band.astype(jnp.int32)
    img_band = imgp_ref[0, :, pl.ds(y0, band_h), :]

    accw = jnp.zeros((th, w_out), jnp.float32)
    acci = jnp.zeros((3, th, w_out), jnp.float32)
    accd = jnp.full((th, w_out), -1, jnp.int32)
    by_ox = {}
    for off, d32, c32, need_min in _TAPS:
        by_ox.setdefault(off % _WP, []).append((off // _WP, d32, c32, need_min))
    for ox, col_taps in sorted(by_ox.items()):
        r_x = r[:, ox:ox + w_out]
        rcp_x = rcp[:, ox:ox + w_out]
        fdi_x = fdi[:, ox:ox + w_out]
        img_x = img_band[:, :, ox:ox + w_out]
        for oy, d32, c32, need_min in col_taps:
            rs = r_x[oy:oy + th, :]
            t = rs * c32 if c32 != np.float32(1.0) else rs
            t = t + jnp.float32(0.5)
            t = t - d32
            ws = jnp.maximum(t, jnp.float32(0.0))
            if need_min:
                ws = jnp.minimum(ws, jnp.float32(1.0))
            w = ws * rcp_x[oy:oy + th, :]
            accw = accw + w
            acci = acci + w[None, :, :] * img_x[:, oy:oy + th, :]
            accd = jnp.maximum(accd, jnp.where(t > jnp.float32(0.0),
                                               fdi_x[oy:oy + th, :], -1))
    bokeh_ref[0, :, :, :] = acci / accw[None]
    dil_ref[0, 0, :, :] = accd.astype(jnp.float32)


def kernel(image, defocus):
    bsz, c, h, w = image.shape
    imgp = jnp.pad(image, ((0, 0), (0, 0), (_R, _HP - h - _R), (_R, _WP - w - _R)))
    dp = jnp.pad(defocus, ((0, 0), (0, 0), (_R, _HP - h - _R), (_R, _WP - w - _R)))

    mesh = plsc.VectorSubcoreMesh(
        core_axis_name="c", subcore_axis_name="s", num_cores=2, num_subcores=16)
    f = pl.kernel(
        _sc_body,
        out_type=[
            jax.ShapeDtypeStruct((bsz * c * _YSC * w,), jnp.float32),
            jax.ShapeDtypeStruct((bsz * _YSC * w,), jnp.float32),
        ],
        mesh=mesh,
        scratch_types=[
            pltpu.VMEM((_BSZ,), jnp.float32),
            pltpu.VMEM((_BSZ,), jnp.float32),
            pltpu.VMEM((_BSZ,), jnp.float32),
            pltpu.VMEM((_BSZ,), jnp.float32),
            pltpu.VMEM((_BSZ,), jnp.float32),
            pltpu.VMEM((3 * _OSZ,), jnp.float32),
            pltpu.VMEM((_OSZ,), jnp.float32),
        ],
        compiler_params=pltpu.CompilerParams(
            use_tc_tiling_on_sc=False, needs_layout_passes=False),
    )
    bok_sc, dil_sc = f(imgp.reshape(bsz * c * _PLANE), dp.reshape(bsz * _PLANE))
    bok_sc = bok_sc.reshape(bsz, c, _YSC, w)
    dil_sc = dil_sc.reshape(bsz, 1, _YSC, w)

    htc = h - _YSC
    bok_tc, dil_tc = pl.pallas_call(
        _tc_body,
        grid=(bsz, htc // _TH),
        in_specs=[
            pl.BlockSpec((1, c, _HP, _WP), lambda bb, yy: (bb, 0, 0, 0)),
            pl.BlockSpec((1, 1, _HP, _WP), lambda bb, yy: (bb, 0, 0, 0)),
        ],
        out_specs=[
            pl.BlockSpec((1, c, _TH, w), lambda bb, yy: (bb, 0, yy, 0)),
            pl.BlockSpec((1, 1, _TH, w), lambda bb, yy: (bb, 0, yy, 0)),
        ],
        out_shape=[
            jax.ShapeDtypeStruct((bsz, c, htc, w), jnp.float32),
            jax.ShapeDtypeStruct((bsz, 1, htc, w), jnp.float32),
        ],
        compiler_params=pltpu.CompilerParams(
            vmem_limit_bytes=100 * 1024 * 1024,
        ),
    )(imgp, dp)

    bokeh = jnp.concatenate([bok_sc, bok_tc], axis=2)
    dil = jnp.concatenate([dil_sc, dil_tc], axis=2)
    return (bokeh, dil)


# speedup vs baseline: 1.0697x; 1.0697x over previous
import numpy as np
import jax
import jax.numpy as jnp
from jax import lax
from jax.experimental import pallas as pl
from jax.experimental.pallas import tpu as pltpu
from jax.experimental.pallas import tpu_sc as plsc

_R = 5
_RC = 8
_BANDL = 24
_W = 512
_WP = 528
_HP = 528
_PLANE = _HP * _WP
_BSZ = _BANDL * _WP
_OSZ = _RC * _W
_YSC = 320
_TH = 32
_TSUB = 8
_RPW = _YSC // 8


def _make_taps():
    m = 2.0 * np.pi / 10000.0
    half = np.pi / 10000.0
    cos_half = np.cos(half)
    init_angle = 3.1415926536 / 2.0
    taps = []
    for dy in range(-_R, _R + 1):
        for dx in range(-_R, _R + 1):
            if dy * dy + dx * dx > 20:
                continue
            dist = np.sqrt(float(dy * dy + dx * dx))
            theta = np.arctan2(float(dy), float(dx)) - init_angle
            ang = np.mod(theta, m) - half
            c32 = np.float32(cos_half / np.cos(ang))
            d32 = np.float32(dist)
            need_min = dy * dy + dx * dx <= 12
            off = int((_R - dy) * _WP + (_R - dx))
            taps.append((off, d32, c32, need_min))
    return taps


_TAPS = _make_taps()


def _sc_body(imgp, dp, bok_out, dil_out, dbuf, rcpbuf, ib0, ib1, ib2, obuf,
             dilbuf):
    ibufs = (ib0, ib1, ib2)
    wid = lax.axis_index("s") * 2 + lax.axis_index("c")
    b = wid // 8
    y0 = (wid % 8) * _RPW
    lanes = lax.broadcasted_iota(jnp.int32, (16,), 0)

    def chunk(k, _):
        ys = y0 + k * _RC

        pltpu.sync_copy(dp.at[pl.ds(b * _PLANE + ys * _WP, _BSZ)], dbuf)
        for cc in range(3):
            pltpu.sync_copy(
                imgp.at[pl.ds((b * 3 + cc) * _PLANE + ys * _WP, _BSZ)],
                ibufs[cc])

        def pre(i, _):
            off = pl.multiple_of(i * 16, 16)
            v = dbuf[pl.ds(off, 16)]
            r = jnp.abs(v)
            dbuf[pl.ds(off, 16)] = r
            rcpbuf[pl.ds(off, 16)] = 1.0 / (r * r + jnp.float32(0.2))
            return 0

        lax.fori_loop(0, (_RC + 2 * _R) * _WP // 16, pre, 0, unroll=False)

        def out_row(yo, _):
            @plsc.parallel_loop(0, _W // 16, unroll=2)
            def group(g):
                c0 = g * 16
                base = yo * _WP + c0
                vbase = jnp.full((16,), base, jnp.int32) + lanes
                vb = [vbase + r for r in range(8)]
                accw = jnp.zeros((16,), jnp.float32)
                acc = [jnp.zeros((16,), jnp.float32) for _ in range(3)]
                mr = jnp.full((16,), -1.0, jnp.float32)
                for off, d32, c32, need_min in _TAPS:
                    off8 = (off // 8) * 8
                    sl = pl.ds(off8, _BSZ - off8)
                    vidx = vb[off % 8]
                    rs = plsc.load_gather(dbuf.at[sl], [vidx])
                    t = rs * c32 if c32 != np.float32(1.0) else rs
                    t = t + jnp.float32(0.5)
                    t = t - d32
                    ws = jnp.maximum(t, jnp.float32(0.0))
                    if need_min:
                        ws = jnp.minimum(ws, jnp.float32(1.0))
                    w = ws * plsc.load_gather(rcpbuf.at[sl], [vidx])
                    accw = accw + w
                    for cc in range(3):
                        acc[cc] = acc[cc] + w * plsc.load_gather(
                            ibufs[cc].at[sl], [vidx])
                    mr = jnp.maximum(
                        mr, jnp.where(t > jnp.float32(0.0), rs,
                                      jnp.float32(-1.0)))
                rw = 1.0 / accw
                oix = pl.multiple_of(yo * _W + c0, 16)
                for cc in range(3):
                    obuf[pl.ds(cc * _OSZ + oix, 16)] = acc[cc] * rw
                dilbuf[pl.ds(oix, 16)] = (
                    mr.astype(jnp.int32).astype(jnp.float32))

            return 0

        lax.fori_loop(0, _RC, out_row, 0, unroll=False)

        for cc in range(3):
            pltpu.sync_copy(
                obuf.at[pl.ds(cc * _OSZ, _OSZ)],
                bok_out.at[pl.ds((b * 3 + cc) * (_YSC * _W) + ys * _W, _OSZ)])
        pltpu.sync_copy(
            dilbuf, dil_out.at[pl.ds(b * (_YSC * _W) + ys * _W, _OSZ)])
        return 0

    lax.fori_loop(0, _RPW // _RC, chunk, 0, unroll=False)


def _tc_body(imgp_ref, dp_ref, bokeh_ref, dil_ref):
    th = bokeh_ref.shape[2]
    w_out = bokeh_ref.shape[3]
    y0 = (pl.program_id(1) + _YSC // _TH) * th
    band_h = th + 2 * _R
    d_band = dp_ref[0, 0, pl.ds(y0, band_h), :]
    r = jnp.abs(d_band)
    rcp = 1.0 / (r * r + jnp.float32(0.2))
    img_band = imgp_ref[0, :, pl.ds(y0, band_h), :]

    accw = jnp.zeros((th, w_out), jnp.float32)
    acci = jnp.zeros((3, th, w_out), jnp.float32)
    mr = jnp.full((th, w_out), -1.0, jnp.float32)
    for off, d32, c32, need_min in _TAPS:
        oy, ox = off // _WP, off % _WP
        rs = r[oy:oy + th, ox:ox + w_out]
        t = rs * c32 if c32 != np.float32(1.0) else rs
        t = t + jnp.float32(0.5)
        t = t - d32
        ws = jnp.maximum(t, jnp.float32(0.0))
        if need_min:
            ws = jnp.minimum(ws, jnp.float32(1.0))
        w = ws * rcp[oy:oy + th, ox:ox + w_out]
        accw = accw + w
        acci = acci + w[None, :, :] * img_band[:, oy:oy + th, ox:ox + w_out]
        mr = jnp.maximum(mr, jnp.where(t > jnp.float32(0.0), rs,
                                       jnp.float32(-1.0)))
    bokeh_ref[0, :, :, :] = acci / accw[None]
    dil_ref[0, 0, :, :] = mr.astype(jnp.int32).astype(jnp.float32)


def kernel(image, defocus):
    bsz, c, h, w = image.shape
    imgp = jnp.pad(image, ((0, 0), (0, 0), (_R, _HP - h - _R), (_R, _WP - w - _R)))
    dp = jnp.pad(defocus, ((0, 0), (0, 0), (_R, _HP - h - _R), (_R, _WP - w - _R)))

    mesh = plsc.VectorSubcoreMesh(
        core_axis_name="c", subcore_axis_name="s", num_cores=2, num_subcores=16)
    f = pl.kernel(
        _sc_body,
        out_type=[
            jax.ShapeDtypeStruct((bsz * c * _YSC * w,), jnp.float32),
            jax.ShapeDtypeStruct((bsz * _YSC * w,), jnp.float32),
        ],
        mesh=mesh,
        scratch_types=[
            pltpu.VMEM((_BSZ,), jnp.float32),
            pltpu.VMEM((_BSZ,), jnp.float32),
            pltpu.VMEM((_BSZ,), jnp.float32),
            pltpu.VMEM((_BSZ,), jnp.float32),
            pltpu.VMEM((_BSZ,), jnp.float32),
            pltpu.VMEM((3 * _OSZ,), jnp.float32),
            pltpu.VMEM((_OSZ,), jnp.float32),
        ],
        compiler_params=pltpu.CompilerParams(
            use_tc_tiling_on_sc=False, needs_layout_passes=False),
    )
    bok_sc, dil_sc = f(imgp.reshape(bsz * c * _PLANE), dp.reshape(bsz * _PLANE))
    bok_sc = bok_sc.reshape(bsz, c, _YSC, w)
    dil_sc = dil_sc.reshape(bsz, 1, _YSC, w)

    htc = h - _YSC
    bok_tc, dil_tc = pl.pallas_call(
        _tc_body,
        grid=(bsz, htc // _TH),
        in_specs=[
            pl.BlockSpec((1, c, _HP, _WP), lambda bb, yy: (bb, 0, 0, 0)),
            pl.BlockSpec((1, 1, _HP, _WP), lambda bb, yy: (bb, 0, 0, 0)),
        ],
        out_specs=[
            pl.BlockSpec((1, c, _TH, w), lambda bb, yy: (bb, 0, yy, 0)),
            pl.BlockSpec((1, 1, _TH, w), lambda bb, yy: (bb, 0, yy, 0)),
        ],
        out_shape=[
            jax.ShapeDtypeStruct((bsz, c, htc, w), jnp.float32),
            jax.ShapeDtypeStruct((bsz, 1, htc, w), jnp.float32),
        ],
        compiler_params=pltpu.CompilerParams(
            vmem_limit_bytes=100 * 1024 * 1024,
        ),
    )(imgp, dp)

    bokeh = jnp.concatenate([bok_sc, bok_tc], axis=2)
    dil = jnp.concatenate([dil_sc, dil_tc], axis=2)
    return (bokeh, dil)


# speedup vs baseline: 1.1917x; 1.1141x over previous
import numpy as np
import jax
import jax.numpy as jnp
from jax import lax
from jax.experimental import pallas as pl
from jax.experimental.pallas import tpu as pltpu
from jax.experimental.pallas import tpu_sc as plsc

_R = 5
_RC = 8
_BANDL = 24
_W = 512
_WP = 528
_HP = 528
_PLANE = _HP * _WP
_BSZ = _BANDL * _WP
_OSZ = _RC * _W
_YSC = 256
_TH = 64
_TSUB = 8
_RPW = _YSC // 8


def _make_taps():
    m = 2.0 * np.pi / 10000.0
    half = np.pi / 10000.0
    cos_half = np.cos(half)
    init_angle = 3.1415926536 / 2.0
    taps = []
    for dy in range(-_R, _R + 1):
        for dx in range(-_R, _R + 1):
            if dy * dy + dx * dx > 20:
                continue
            dist = np.sqrt(float(dy * dy + dx * dx))
            theta = np.arctan2(float(dy), float(dx)) - init_angle
            ang = np.mod(theta, m) - half
            c32 = np.float32(cos_half / np.cos(ang))
            d32 = np.float32(dist)
            need_min = dy * dy + dx * dx <= 12
            off = int((_R - dy) * _WP + (_R - dx))
            taps.append((off, d32, c32, need_min))
    return taps


_TAPS = _make_taps()


def _sc_body(imgp, dp, bok_out, dil_out, dbuf, rcpbuf, ib0, ib1, ib2, obuf,
             dilbuf):
    ibufs = (ib0, ib1, ib2)
    wid = lax.axis_index("s") * 2 + lax.axis_index("c")
    b = wid // 8
    y0 = (wid % 8) * _RPW
    lanes = lax.broadcasted_iota(jnp.int32, (16,), 0)

    def chunk(k, _):
        ys = y0 + k * _RC

        pltpu.sync_copy(dp.at[pl.ds(b * _PLANE + ys * _WP, _BSZ)], dbuf)
        for cc in range(3):
            pltpu.sync_copy(
                imgp.at[pl.ds((b * 3 + cc) * _PLANE + ys * _WP, _BSZ)],
                ibufs[cc])

        def pre(i, _):
            off = pl.multiple_of(i * 16, 16)
            v = dbuf[pl.ds(off, 16)]
            r = jnp.abs(v)
            dbuf[pl.ds(off, 16)] = r
            rcpbuf[pl.ds(off, 16)] = 1.0 / (r * r + jnp.float32(0.2))
            return 0

        lax.fori_loop(0, (_RC + 2 * _R) * _WP // 16, pre, 0, unroll=False)

        def out_row(yo, _):
            @plsc.parallel_loop(0, _W // 16, unroll=2)
            def group(g):
                c0 = g * 16
                base = yo * _WP + c0
                vbase = jnp.full((16,), base, jnp.int32) + lanes
                vb = [vbase + r for r in range(8)]
                accw = jnp.zeros((16,), jnp.float32)
                acc = [jnp.zeros((16,), jnp.float32) for _ in range(3)]
                mr = jnp.full((16,), -1.0, jnp.float32)
                for off, d32, c32, need_min in _TAPS:
                    off8 = (off // 8) * 8
                    sl = pl.ds(off8, _BSZ - off8)
                    vidx = vb[off % 8]
                    rs = plsc.load_gather(dbuf.at[sl], [vidx])
                    t = rs * c32 if c32 != np.float32(1.0) else rs
                    t = t + jnp.float32(0.5)
                    t = t - d32
                    ws = jnp.maximum(t, jnp.float32(0.0))
                    if need_min:
                        ws = jnp.minimum(ws, jnp.float32(1.0))
                    w = ws * plsc.load_gather(rcpbuf.at[sl], [vidx])
                    accw = accw + w
                    for cc in range(3):
                        acc[cc] = acc[cc] + w * plsc.load_gather(
                            ibufs[cc].at[sl], [vidx])
                    mr = jnp.maximum(
                        mr, jnp.where(t > jnp.float32(0.0), rs,
                                      jnp.float32(-1.0)))
                rw = 1.0 / accw
                oix = pl.multiple_of(yo * _W + c0, 16)
                for cc in range(3):
                    obuf[pl.ds(cc * _OSZ + oix, 16)] = acc[cc] * rw
                dilbuf[pl.ds(oix, 16)] = (
                    mr.astype(jnp.int32).astype(jnp.float32))

            return 0

        lax.fori_loop(0, _RC, out_row, 0, unroll=False)

        for cc in range(3):
            pltpu.sync_copy(
                obuf.at[pl.ds(cc * _OSZ, _OSZ)],
                bok_out.at[pl.ds((b * 3 + cc) * (_YSC * _W) + ys * _W, _OSZ)])
        pltpu.sync_copy(
            dilbuf, dil_out.at[pl.ds(b * (_YSC * _W) + ys * _W, _OSZ)])
        return 0

    lax.fori_loop(0, _RPW // _RC, chunk, 0, unroll=False)


def _tc_body(imgp_ref, dp_ref, bokeh_ref, dil_ref):
    th = bokeh_ref.shape[2]
    w_out = bokeh_ref.shape[3]
    y0 = (pl.program_id(1) + _YSC // _TH) * th
    band_h = th + 2 * _R
    d_band = dp_ref[0, 0, pl.ds(y0, band_h), :]
    r = jnp.abs(d_band)
    rcp = 1.0 / (r * r + jnp.float32(0.2))
    img_band = imgp_ref[0, :, pl.ds(y0, band_h), :]

    accw = jnp.zeros((th, w_out), jnp.float32)
    acci = jnp.zeros((3, th, w_out), jnp.float32)
    mr = jnp.full((th, w_out), -1.0, jnp.float32)
    for off, d32, c32, need_min in _TAPS:
        oy, ox = off // _WP, off % _WP
        rs = r[oy:oy + th, ox:ox + w_out]
        t = rs * c32 if c32 != np.float32(1.0) else rs
        t = t + jnp.float32(0.5)
        t = t - d32
        ws = jnp.maximum(t, jnp.float32(0.0))
        if need_min:
            ws = jnp.minimum(ws, jnp.float32(1.0))
        w = ws * rcp[oy:oy + th, ox:ox + w_out]
        accw = accw + w
        acci = acci + w[None, :, :] * img_band[:, oy:oy + th, ox:ox + w_out]
        mr = jnp.maximum(mr, jnp.where(t > jnp.float32(0.0), rs,
                                       jnp.float32(-1.0)))
    bokeh_ref[0, :, :, :] = acci / accw[None]
    dil_ref[0, 0, :, :] = mr.astype(jnp.int32).astype(jnp.float32)


def kernel(image, defocus):
    bsz, c, h, w = image.shape
    imgp = jnp.pad(image, ((0, 0), (0, 0), (_R, _HP - h - _R), (_R, _WP - w - _R)))
    dp = jnp.pad(defocus, ((0, 0), (0, 0), (_R, _HP - h - _R), (_R, _WP - w - _R)))

    mesh = plsc.VectorSubcoreMesh(
        core_axis_name="c", subcore_axis_name="s", num_cores=2, num_subcores=16)
    f = pl.kernel(
        _sc_body,
        out_type=[
            jax.ShapeDtypeStruct((bsz * c * _YSC * w,), jnp.float32),
            jax.ShapeDtypeStruct((bsz * _YSC * w,), jnp.float32),
        ],
        mesh=mesh,
        scratch_types=[
            pltpu.VMEM((_BSZ,), jnp.float32),
            pltpu.VMEM((_BSZ,), jnp.float32),
            pltpu.VMEM((_BSZ,), jnp.float32),
            pltpu.VMEM((_BSZ,), jnp.float32),
            pltpu.VMEM((_BSZ,), jnp.float32),
            pltpu.VMEM((3 * _OSZ,), jnp.float32),
            pltpu.VMEM((_OSZ,), jnp.float32),
        ],
        compiler_params=pltpu.CompilerParams(
            use_tc_tiling_on_sc=False, needs_layout_passes=False),
    )
    bok_sc, dil_sc = f(imgp.reshape(bsz * c * _PLANE), dp.reshape(bsz * _PLANE))
    bok_sc = bok_sc.reshape(bsz, c, _YSC, w)
    dil_sc = dil_sc.reshape(bsz, 1, _YSC, w)

    htc = h - _YSC
    bok_tc, dil_tc = pl.pallas_call(
        _tc_body,
        grid=(bsz, htc // _TH),
        in_specs=[
            pl.BlockSpec((1, c, _HP, _WP), lambda bb, yy: (bb, 0, 0, 0)),
            pl.BlockSpec((1, 1, _HP, _WP), lambda bb, yy: (bb, 0, 0, 0)),
        ],
        out_specs=[
            pl.BlockSpec((1, c, _TH, w), lambda bb, yy: (bb, 0, yy, 0)),
            pl.BlockSpec((1, 1, _TH, w), lambda bb, yy: (bb, 0, yy, 0)),
        ],
        out_shape=[
            jax.ShapeDtypeStruct((bsz, c, htc, w), jnp.float32),
            jax.ShapeDtypeStruct((bsz, 1, htc, w), jnp.float32),
        ],
        compiler_params=pltpu.CompilerParams(
            vmem_limit_bytes=100 * 1024 * 1024,
        ),
    )(imgp, dp)

    bokeh = jnp.concatenate([bok_sc, bok_tc], axis=2)
    dil = jnp.concatenate([dil_sc, dil_tc], axis=2)
    return (bokeh, dil)
